# baseline (speedup 1.0000x reference)
import functools

import jax
import jax.numpy as jnp
from jax import lax
from jax.experimental import pallas as pl
from jax.experimental.pallas import tpu as pltpu
from jax.experimental.pallas import tpu_sc as plsc

N = 10000
E = 320000
D = 128

NC, NS = 2, 16
NW = NC * NS
CHUNK = 128
SCH = 20
NSCH = E // (CHUNK * SCH)
MAXS = 4
N_ACC = 10112
RPT = N_ACC // NS

_mesh = plsc.VectorSubcoreMesh(core_axis_name="c", subcore_axis_name="s")


def _worker_window(wid):
    sc0 = (wid * NSCH) // NW
    scnt = ((wid + 1) * NSCH) // NW - sc0
    return sc0, scnt


def _zero_acc(zbuf, acc, sid):
    nfull = RPT // CHUNK
    rem = RPT - nfull * CHUNK

    def _zero(j, _):
        pltpu.sync_copy(zbuf, acc.at[pl.ds(sid * RPT + j * CHUNK, CHUNK)])
        return 0
    lax.fori_loop(0, nfull, _zero, 0)
    if rem:
        pltpu.sync_copy(zbuf.at[pl.ds(0, rem)],
                        acc.at[pl.ds(sid * RPT + nfull * CHUNK, rem)])


def _load_idx(ei_hbm, which, sc0, scnt, dst):
    def _one(k, _):
        pltpu.sync_copy(ei_hbm.at[which, sc0 + k], dst.at[k])
        return 0
    lax.fori_loop(0, scnt, _one, 0)


@functools.partial(
    pl.kernel,
    out_type=jax.ShapeDtypeStruct((NC, N_ACC, 16), jnp.float32),
    mesh=_mesh,
    scratch_types=[
        pltpu.VMEM((MAXS, SCH, CHUNK), jnp.int32),
        pltpu.VMEM((CHUNK, 16), jnp.float32),
        pltpu.VMEM((CHUNK, 16), jnp.float32),
        pltpu.SemaphoreType.DMA,
        pltpu.VMEM_SHARED((N_ACC, 16), jnp.float32),
    ],
)
def _sc_bincount(ei_hbm, deg_out, row_idx_v, ones_v, zeros_v, asem, deg_acc):
    cid = lax.axis_index("c")
    sid = lax.axis_index("s")
    wid = sid * NC + cid
    sc0, scnt = _worker_window(wid)
    nchunks = scnt * SCH

    def _fill(i, _):
        ones_v[i] = jnp.ones((16,), jnp.float32)
        zeros_v[i] = jnp.zeros((16,), jnp.float32)
        return 0
    lax.fori_loop(0, CHUNK, _fill, 0)

    _zero_acc(zeros_v, deg_acc, sid)
    plsc.subcore_barrier()

    _load_idx(ei_hbm, 0, sc0, scnt, row_idx_v)

    def _accum(j, _):
        pltpu.async_copy(ones_v, deg_acc.at[row_idx_v.at[j // SCH, j % SCH]],
                         asem, add=True)
        return 0
    lax.fori_loop(0, nchunks, _accum, 0)

    def _drain(j, _):
        pltpu.make_async_copy(ones_v, deg_acc.at[row_idx_v.at[0, 0]],
                              asem).wait()
        return 0
    lax.fori_loop(0, nchunks, _drain, 0)
    plsc.subcore_barrier()

    pltpu.sync_copy(deg_acc.at[pl.ds(sid * RPT, RPT)],
                    deg_out.at[cid, pl.ds(sid * RPT, RPT)])


def _scale_mm_body(dp_ref, x_ref, w_ref, o_ref):
    deg = (dp_ref[0] + dp_ref[1])[:N]
    dis = lax.rsqrt(deg[:, :1])
    o_ref[:N] = dis * jnp.dot(x_ref[...], w_ref[...],
                              preferred_element_type=jnp.float32)


def _scale_mm(dp, x, weight):
    return pl.pallas_call(
        _scale_mm_body,
        out_shape=jax.ShapeDtypeStruct((N_ACC, D), jnp.float32),
    )(dp, x, weight)


@functools.partial(
    pl.kernel,
    out_type=jax.ShapeDtypeStruct((NC, N_ACC, D), jnp.float32),
    mesh=_mesh,
    scratch_types=[
        pltpu.VMEM((2, SCH, CHUNK), jnp.int32),
        pltpu.VMEM((2, SCH, CHUNK), jnp.int32),
        pltpu.VMEM((2, CHUNK, D), jnp.float32),
        pltpu.SemaphoreType.DMA,
        pltpu.SemaphoreType.DMA,
        pltpu.SemaphoreType.DMA,
        pltpu.VMEM_SHARED((N_ACC, D), jnp.float32),
    ],
)
def _sc_scatter(scaled_hbm, ei_hbm, out_hbm,
                row_idx_v, col_idx_v, rows_v, gsem, isem, ssem, acc):
    cid = lax.axis_index("c")
    sid = lax.axis_index("s")
    wid = sid * NC + cid
    sc0, scnt = _worker_window(wid)

    def _fill(k, _):
        rows_v[0, k // 8, pl.ds((k % 8) * 16, 16)] = jnp.zeros((16,), jnp.float32)
        return 0
    lax.fori_loop(0, CHUNK * D // 16, _fill, 0)

    _zero_acc(rows_v.at[0], acc, sid)
    plsc.subcore_barrier()

    def _wait_gather(cbuf, k, rbuf):
        pltpu.make_async_copy(scaled_hbm.at[cbuf.at[k]], rbuf, gsem).wait()

    def _wait_scatter():
        pltpu.make_async_copy(rows_v.at[0], acc.at[row_idx_v.at[0, 0]],
                              ssem).wait()

    pltpu.sync_copy(ei_hbm.at[0, sc0], row_idx_v.at[0])
    pltpu.sync_copy(ei_hbm.at[1, sc0], col_idx_v.at[0])
    pltpu.async_copy(scaled_hbm.at[col_idx_v.at[0, 0]], rows_v.at[0], gsem)

    for s in range(MAXS):
        @pl.when(s < scnt)
        def _superchunk():
            rix, cix = row_idx_v.at[s % 2], col_idx_v.at[s % 2]

            @pl.when(s + 1 < scnt)
            def _prefetch_idx():
                pltpu.async_copy(ei_hbm.at[0, sc0 + s + 1],
                                 row_idx_v.at[(s + 1) % 2], isem)
                pltpu.async_copy(ei_hbm.at[1, sc0 + s + 1],
                                 col_idx_v.at[(s + 1) % 2], isem)

            def _chunk(k, _):
                _wait_gather(cix, k, rows_v.at[k % 2])
                pltpu.async_copy(rows_v.at[k % 2], acc.at[rix.at[k]], ssem,
                                 add=True)
                if s == 0:
                    @pl.when(k > 0)
                    def _():
                        _wait_scatter()
                else:
                    _wait_scatter()
                pltpu.async_copy(scaled_hbm.at[cix.at[k + 1]],
                                 rows_v.at[(k + 1) % 2], gsem)
                return 0
            lax.fori_loop(0, SCH - 1, _chunk, 0)

            _wait_gather(cix, SCH - 1, rows_v.at[(SCH - 1) % 2])
            pltpu.async_copy(rows_v.at[(SCH - 1) % 2],
                             acc.at[rix.at[SCH - 1]], ssem, add=True)
            _wait_scatter()

            @pl.when(s + 1 < scnt)
            def _fire_next():
                pltpu.make_async_copy(ei_hbm.at[0, sc0],
                                      row_idx_v.at[(s + 1) % 2], isem).wait()
                pltpu.make_async_copy(ei_hbm.at[1, sc0],
                                      col_idx_v.at[(s + 1) % 2], isem).wait()
                pltpu.async_copy(
                    scaled_hbm.at[col_idx_v.at[(s + 1) % 2, 0]],
                    rows_v.at[0], gsem)

    _wait_scatter()
    plsc.subcore_barrier()
    pltpu.sync_copy(acc.at[pl.ds(sid * RPT, RPT)],
                    out_hbm.at[cid, pl.ds(sid * RPT, RPT)])


def _final_body(dp_ref, ap_ref, b_ref, o_ref):
    deg = (dp_ref[0] + dp_ref[1])[:, :1]
    dis = jnp.where(deg > 0, lax.rsqrt(deg), 0.0)
    o_ref[...] = dis * (ap_ref[0] + ap_ref[1]) + b_ref[...]


def _finalize(dp, ap, bias2d):
    blk = 1000
    grid = (N // blk,)
    return pl.pallas_call(
        _final_body,
        grid=grid,
        in_specs=[
            pl.BlockSpec((NC, blk, 16), lambda i: (0, i, 0)),
            pl.BlockSpec((NC, blk, D), lambda i: (0, i, 0)),
            pl.BlockSpec((1, D), lambda i: (0, 0)),
        ],
        out_specs=pl.BlockSpec((blk, D), lambda i: (i, 0)),
        out_shape=jax.ShapeDtypeStruct((N, D), jnp.float32),
    )(dp, ap, bias2d)


def kernel(x, edge_index, weight, bias):
    ei = edge_index.astype(jnp.int32).reshape(2, NSCH, SCH, CHUNK)

    dp = _sc_bincount(ei)
    scaled = _scale_mm(dp, x, weight)
    ap = _sc_scatter(scaled, ei)
    return _finalize(dp, ap, bias.reshape(1, D))


# speedup vs baseline: 1.0077x; 1.0077x over previous
import functools

import jax
import jax.numpy as jnp
from jax import lax
from jax.experimental import pallas as pl
from jax.experimental.pallas import tpu as pltpu
from jax.experimental.pallas import tpu_sc as plsc

N = 10000
E = 320000
D = 128

NC, NS = 2, 16
NW = NC * NS
CHUNK = 128
SCH = 20
NSCH = E // (CHUNK * SCH)
MAXS = 4
N_ACC = 10112
RPT = N_ACC // NS

_mesh = plsc.VectorSubcoreMesh(core_axis_name="c", subcore_axis_name="s")


def _worker_window(wid):
    sc0 = (wid * NSCH) // NW
    scnt = ((wid + 1) * NSCH) // NW - sc0
    return sc0, scnt


def _zero_acc(zbuf, acc, sid):
    nfull = RPT // CHUNK
    rem = RPT - nfull * CHUNK

    def _zero(j, _):
        pltpu.sync_copy(zbuf, acc.at[pl.ds(sid * RPT + j * CHUNK, CHUNK)])
        return 0
    lax.fori_loop(0, nfull, _zero, 0)
    if rem:
        pltpu.sync_copy(zbuf.at[pl.ds(0, rem)],
                        acc.at[pl.ds(sid * RPT + nfull * CHUNK, rem)])


def _load_idx(ei_hbm, which, sc0, scnt, dst):
    def _one(k, _):
        pltpu.sync_copy(ei_hbm.at[which, sc0 + k], dst.at[k])
        return 0
    lax.fori_loop(0, scnt, _one, 0)


DW = 16

@functools.partial(
    pl.kernel,
    out_type=jax.ShapeDtypeStruct((NC, N_ACC, DW), jnp.float32),
    mesh=_mesh,
    scratch_types=[
        pltpu.VMEM((MAXS, SCH, CHUNK), jnp.int32),
        pltpu.VMEM((CHUNK, DW), jnp.float32),
        pltpu.VMEM((CHUNK, DW), jnp.float32),
        pltpu.SemaphoreType.DMA,
        pltpu.VMEM_SHARED((N_ACC, DW), jnp.float32),
    ],
)
def _sc_bincount(ei_hbm, deg_out, row_idx_v, ones_v, zeros_v, asem, deg_acc):
    cid = lax.axis_index("c")
    sid = lax.axis_index("s")
    wid = sid * NC + cid
    sc0, scnt = _worker_window(wid)
    nchunks = scnt * SCH

    def _fill(i, _):
        ones_v[i] = jnp.ones((DW,), jnp.float32)
        zeros_v[i] = jnp.zeros((DW,), jnp.float32)
        return 0
    lax.fori_loop(0, CHUNK, _fill, 0)

    _load_idx(ei_hbm, 0, sc0, scnt, row_idx_v)

    _zero_acc(zeros_v, deg_acc, sid)
    plsc.subcore_barrier()

    def _accum(j, _):
        pltpu.async_copy(ones_v, deg_acc.at[row_idx_v.at[j // SCH, j % SCH]],
                         asem, add=True)
        return 0
    lax.fori_loop(0, nchunks, _accum, 0)

    def _drain(j, _):
        pltpu.make_async_copy(ones_v, deg_acc.at[row_idx_v.at[0, 0]],
                              asem).wait()
        return 0
    lax.fori_loop(0, nchunks, _drain, 0)
    plsc.subcore_barrier()

    pltpu.sync_copy(deg_acc.at[pl.ds(sid * RPT, RPT)],
                    deg_out.at[cid, pl.ds(sid * RPT, RPT)])


def _scale_mm_body(dp_ref, x_ref, w_ref, o_ref):
    deg = (dp_ref[0] + dp_ref[1])[:N]
    dis = lax.rsqrt(deg[:, :1])
    o_ref[:N] = dis * jnp.dot(x_ref[...], w_ref[...],
                              preferred_element_type=jnp.float32)


def _scale_mm(dp, x, weight):
    return pl.pallas_call(
        _scale_mm_body,
        out_shape=jax.ShapeDtypeStruct((N_ACC, D), jnp.float32),
    )(dp, x, weight)


@functools.partial(
    pl.kernel,
    out_type=jax.ShapeDtypeStruct((NC, N_ACC, D), jnp.float32),
    mesh=_mesh,
    scratch_types=[
        pltpu.VMEM((2, SCH, CHUNK), jnp.int32),
        pltpu.VMEM((2, SCH, CHUNK), jnp.int32),
        pltpu.VMEM((2, CHUNK, D), jnp.float32),
        pltpu.SemaphoreType.DMA,
        pltpu.SemaphoreType.DMA,
        pltpu.SemaphoreType.DMA,
        pltpu.VMEM_SHARED((N_ACC, D), jnp.float32),
    ],
)
def _sc_scatter(scaled_hbm, ei_hbm, out_hbm,
                row_idx_v, col_idx_v, rows_v, gsem, isem, ssem, acc):
    cid = lax.axis_index("c")
    sid = lax.axis_index("s")
    wid = sid * NC + cid
    sc0, scnt = _worker_window(wid)

    def _fill(k, _):
        rows_v[0, k // 8, pl.ds((k % 8) * 16, 16)] = jnp.zeros((16,), jnp.float32)
        return 0
    lax.fori_loop(0, CHUNK * D // 16, _fill, 0)

    _zero_acc(rows_v.at[0], acc, sid)
    plsc.subcore_barrier()

    def _wait_gather(cbuf, k, rbuf):
        pltpu.make_async_copy(scaled_hbm.at[cbuf.at[k]], rbuf, gsem).wait()

    def _wait_scatter():
        pltpu.make_async_copy(rows_v.at[0], acc.at[row_idx_v.at[0, 0]],
                              ssem).wait()

    pltpu.sync_copy(ei_hbm.at[0, sc0], row_idx_v.at[0])
    pltpu.sync_copy(ei_hbm.at[1, sc0], col_idx_v.at[0])
    pltpu.async_copy(scaled_hbm.at[col_idx_v.at[0, 0]], rows_v.at[0], gsem)

    for s in range(MAXS):
        @pl.when(s < scnt)
        def _superchunk():
            rix, cix = row_idx_v.at[s % 2], col_idx_v.at[s % 2]

            @pl.when(s + 1 < scnt)
            def _prefetch_idx():
                pltpu.async_copy(ei_hbm.at[0, sc0 + s + 1],
                                 row_idx_v.at[(s + 1) % 2], isem)
                pltpu.async_copy(ei_hbm.at[1, sc0 + s + 1],
                                 col_idx_v.at[(s + 1) % 2], isem)

            def _chunk(k, _):
                _wait_gather(cix, k, rows_v.at[k % 2])
                pltpu.async_copy(rows_v.at[k % 2], acc.at[rix.at[k]], ssem,
                                 add=True)
                if s == 0:
                    @pl.when(k > 0)
                    def _():
                        _wait_scatter()
                else:
                    _wait_scatter()
                pltpu.async_copy(scaled_hbm.at[cix.at[k + 1]],
                                 rows_v.at[(k + 1) % 2], gsem)
                return 0
            lax.fori_loop(0, SCH - 1, _chunk, 0)

            _wait_gather(cix, SCH - 1, rows_v.at[(SCH - 1) % 2])
            pltpu.async_copy(rows_v.at[(SCH - 1) % 2],
                             acc.at[rix.at[SCH - 1]], ssem, add=True)
            _wait_scatter()

            @pl.when(s + 1 < scnt)
            def _fire_next():
                pltpu.make_async_copy(ei_hbm.at[0, sc0],
                                      row_idx_v.at[(s + 1) % 2], isem).wait()
                pltpu.make_async_copy(ei_hbm.at[1, sc0],
                                      col_idx_v.at[(s + 1) % 2], isem).wait()
                pltpu.async_copy(
                    scaled_hbm.at[col_idx_v.at[(s + 1) % 2, 0]],
                    rows_v.at[0], gsem)

    _wait_scatter()
    plsc.subcore_barrier()
    pltpu.sync_copy(acc.at[pl.ds(sid * RPT, RPT)],
                    out_hbm.at[cid, pl.ds(sid * RPT, RPT)])


def _final_body(dp_ref, ap_ref, b_ref, o_ref):
    deg = (dp_ref[0] + dp_ref[1])[:N, :1]
    dis = jnp.where(deg > 0, lax.rsqrt(deg), 0.0)
    o_ref[...] = dis * (ap_ref[0] + ap_ref[1])[:N] + b_ref[...]


def _finalize(dp, ap, bias2d):
    return pl.pallas_call(
        _final_body,
        out_shape=jax.ShapeDtypeStruct((N, D), jnp.float32),
    )(dp, ap, bias2d)


def kernel(x, edge_index, weight, bias):
    ei = edge_index.astype(jnp.int32).reshape(2, NSCH, SCH, CHUNK)

    dp = _sc_bincount(ei)
    scaled = _scale_mm(dp, x, weight)
    ap = _sc_scatter(scaled, ei)
    return _finalize(dp, ap, bias.reshape(1, D))


# speedup vs baseline: 1.0310x; 1.0231x over previous
import functools

import jax
import jax.numpy as jnp
from jax import lax
from jax.experimental import pallas as pl
from jax.experimental.pallas import tpu as pltpu
from jax.experimental.pallas import tpu_sc as plsc

N = 10000
E = 320000
D = 128

NC, NS = 2, 16
NW = NC * NS
CHUNK = 128
SCH = 20
NSCH = E // (CHUNK * SCH)
MAXS = 4
N_ACC = 10112
RPT = N_ACC // NS

_mesh = plsc.VectorSubcoreMesh(core_axis_name="c", subcore_axis_name="s")


def _worker_window(wid):
    sc0 = (wid * NSCH) // NW
    scnt = ((wid + 1) * NSCH) // NW - sc0
    return sc0, scnt


def _zero_acc(zbuf, acc, sid):
    nfull = RPT // CHUNK
    rem = RPT - nfull * CHUNK

    def _zero(j, _):
        pltpu.sync_copy(zbuf, acc.at[pl.ds(sid * RPT + j * CHUNK, CHUNK)])
        return 0
    lax.fori_loop(0, nfull, _zero, 0)
    if rem:
        pltpu.sync_copy(zbuf.at[pl.ds(0, rem)],
                        acc.at[pl.ds(sid * RPT + nfull * CHUNK, rem)])


def _load_idx(ei_hbm, which, sc0, scnt, dst):
    def _one(k, _):
        pltpu.sync_copy(ei_hbm.at[which, sc0 + k], dst.at[k])
        return 0
    lax.fori_loop(0, scnt, _one, 0)


DW = 16

@functools.partial(
    pl.kernel,
    out_type=jax.ShapeDtypeStruct((NC, N_ACC, DW), jnp.float32),
    mesh=_mesh,
    scratch_types=[
        pltpu.VMEM((MAXS, SCH, CHUNK), jnp.int32),
        pltpu.VMEM((CHUNK, DW), jnp.float32),
        pltpu.VMEM((CHUNK, DW), jnp.float32),
        pltpu.SemaphoreType.DMA,
        pltpu.VMEM_SHARED((N_ACC, DW), jnp.float32),
    ],
)
def _sc_bincount(ei_hbm, deg_out, row_idx_v, ones_v, zeros_v, asem, deg_acc):
    cid = lax.axis_index("c")
    sid = lax.axis_index("s")
    wid = sid * NC + cid
    sc0, scnt = _worker_window(wid)
    nchunks = scnt * SCH

    def _fill(i, _):
        ones_v[i] = jnp.ones((DW,), jnp.float32)
        zeros_v[i] = jnp.zeros((DW,), jnp.float32)
        return 0
    lax.fori_loop(0, CHUNK, _fill, 0)

    _load_idx(ei_hbm, 0, sc0, scnt, row_idx_v)

    _zero_acc(zeros_v, deg_acc, sid)
    plsc.subcore_barrier()

    def _accum(j, _):
        pltpu.async_copy(ones_v, deg_acc.at[row_idx_v.at[j // SCH, j % SCH]],
                         asem, add=True)
        return 0
    lax.fori_loop(0, nchunks, _accum, 0)

    def _drain(j, _):
        pltpu.make_async_copy(ones_v, deg_acc.at[row_idx_v.at[0, 0]],
                              asem).wait()
        return 0
    lax.fori_loop(0, nchunks, _drain, 0)
    plsc.subcore_barrier()

    pltpu.sync_copy(deg_acc.at[pl.ds(sid * RPT, RPT)],
                    deg_out.at[cid, pl.ds(sid * RPT, RPT)])


def _scale_mm_body(dp_ref, x_ref, w_ref, o_ref):
    deg = (dp_ref[0] + dp_ref[1])[:N]
    dis = lax.rsqrt(deg[:, :1])
    o_ref[:N] = dis * jnp.dot(x_ref[...], w_ref[...],
                              preferred_element_type=jnp.float32)


def _scale_mm(dp, x, weight):
    return pl.pallas_call(
        _scale_mm_body,
        out_shape=jax.ShapeDtypeStruct((N_ACC, D), jnp.float32),
    )(dp, x, weight)


@functools.partial(
    pl.kernel,
    out_type=jax.ShapeDtypeStruct((NC, N_ACC, D), jnp.float32),
    mesh=_mesh,
    scratch_types=[
        pltpu.VMEM((2, SCH, CHUNK), jnp.int32),
        pltpu.VMEM((2, SCH, CHUNK), jnp.int32),
        pltpu.VMEM((2, CHUNK, D), jnp.float32),
        pltpu.SemaphoreType.DMA,
        pltpu.SemaphoreType.DMA,
        pltpu.SemaphoreType.DMA,
        pltpu.VMEM_SHARED((N_ACC, D), jnp.float32),
    ],
)
def _sc_scatter(scaled_hbm, ei_hbm, out_hbm,
                row_idx_v, col_idx_v, rows_v, gsem, isem, ssem, acc):
    cid = lax.axis_index("c")
    sid = lax.axis_index("s")
    wid = sid * NC + cid
    sc0, scnt = _worker_window(wid)

    def _fill(k, _):
        for c in range(D // 16):
            rows_v[0, k, pl.ds(c * 16, 16)] = jnp.zeros((16,), jnp.float32)
        return 0
    lax.fori_loop(0, CHUNK, _fill, 0)

    nfull = RPT // CHUNK
    rem = RPT - nfull * CHUNK
    for j in range(nfull):
        pltpu.async_copy(rows_v.at[0],
                         acc.at[pl.ds(sid * RPT + j * CHUNK, CHUNK)], ssem)
    pltpu.async_copy(rows_v.at[0, pl.ds(0, rem)],
                     acc.at[pl.ds(sid * RPT + nfull * CHUNK, rem)], ssem)

    pltpu.sync_copy(ei_hbm.at[0, sc0], row_idx_v.at[0])
    pltpu.sync_copy(ei_hbm.at[1, sc0], col_idx_v.at[0])

    for j in range(nfull):
        pltpu.make_async_copy(
            rows_v.at[0], acc.at[pl.ds(sid * RPT, CHUNK)], ssem).wait()
    pltpu.make_async_copy(
        rows_v.at[0, pl.ds(0, rem)], acc.at[pl.ds(sid * RPT, rem)],
        ssem).wait()
    plsc.subcore_barrier()

    def _wait_gather(cbuf, k, rbuf):
        pltpu.make_async_copy(scaled_hbm.at[cbuf.at[k]], rbuf, gsem).wait()

    def _wait_scatter():
        pltpu.make_async_copy(rows_v.at[0], acc.at[row_idx_v.at[0, 0]],
                              ssem).wait()

    pltpu.async_copy(scaled_hbm.at[col_idx_v.at[0, 0]], rows_v.at[0], gsem)

    for s in range(MAXS):
        @pl.when(s < scnt)
        def _superchunk():
            rix, cix = row_idx_v.at[s % 2], col_idx_v.at[s % 2]

            @pl.when(s + 1 < scnt)
            def _prefetch_idx():
                pltpu.async_copy(ei_hbm.at[0, sc0 + s + 1],
                                 row_idx_v.at[(s + 1) % 2], isem)
                pltpu.async_copy(ei_hbm.at[1, sc0 + s + 1],
                                 col_idx_v.at[(s + 1) % 2], isem)

            def _chunk(k, _):
                _wait_gather(cix, k, rows_v.at[k % 2])
                pltpu.async_copy(rows_v.at[k % 2], acc.at[rix.at[k]], ssem,
                                 add=True)
                if s == 0:
                    @pl.when(k > 0)
                    def _():
                        _wait_scatter()
                else:
                    _wait_scatter()
                pltpu.async_copy(scaled_hbm.at[cix.at[k + 1]],
                                 rows_v.at[(k + 1) % 2], gsem)
                return 0
            lax.fori_loop(0, SCH - 1, _chunk, 0)

            _wait_gather(cix, SCH - 1, rows_v.at[(SCH - 1) % 2])
            pltpu.async_copy(rows_v.at[(SCH - 1) % 2],
                             acc.at[rix.at[SCH - 1]], ssem, add=True)
            _wait_scatter()

            @pl.when(s + 1 < scnt)
            def _fire_next():
                pltpu.make_async_copy(ei_hbm.at[0, sc0],
                                      row_idx_v.at[(s + 1) % 2], isem).wait()
                pltpu.make_async_copy(ei_hbm.at[1, sc0],
                                      col_idx_v.at[(s + 1) % 2], isem).wait()
                pltpu.async_copy(
                    scaled_hbm.at[col_idx_v.at[(s + 1) % 2, 0]],
                    rows_v.at[0], gsem)

    _wait_scatter()
    plsc.subcore_barrier()
    pltpu.sync_copy(acc.at[pl.ds(sid * RPT, RPT)],
                    out_hbm.at[cid, pl.ds(sid * RPT, RPT)])


def _final_body(dp_ref, ap_ref, b_ref, o_ref):
    deg = (dp_ref[0] + dp_ref[1])[:N, :1]
    dis = jnp.where(deg > 0, lax.rsqrt(deg), 0.0)
    o_ref[...] = dis * (ap_ref[0] + ap_ref[1])[:N] + b_ref[...]


def _finalize(dp, ap, bias2d):
    return pl.pallas_call(
        _final_body,
        out_shape=jax.ShapeDtypeStruct((N, D), jnp.float32),
    )(dp, ap, bias2d)


def kernel(x, edge_index, weight, bias):
    ei = edge_index.astype(jnp.int32).reshape(2, NSCH, SCH, CHUNK)

    dp = _sc_bincount(ei)
    scaled = _scale_mm(dp, x, weight)
    ap = _sc_scatter(scaled, ei)
    return _finalize(dp, ap, bias.reshape(1, D))


# speedup vs baseline: 1.1683x; 1.1332x over previous
import functools

import jax
import jax.numpy as jnp
from jax import lax
from jax.experimental import pallas as pl
from jax.experimental.pallas import tpu as pltpu
from jax.experimental.pallas import tpu_sc as plsc

N = 10000
E = 320000
D = 128

NC, NS = 2, 16
NW = NC * NS
CHUNK = 128
SCH = 20
NSCH = E // (CHUNK * SCH)
MAXS = 4
N_ACC = 10112
RPT = N_ACC // NS

_mesh = plsc.VectorSubcoreMesh(core_axis_name="c", subcore_axis_name="s")


def _worker_window(wid):
    sc0 = (wid * NSCH) // NW
    scnt = ((wid + 1) * NSCH) // NW - sc0
    return sc0, scnt


def _zero_acc(zbuf, acc, sid):
    nfull = RPT // CHUNK
    rem = RPT - nfull * CHUNK

    def _zero(j, _):
        pltpu.sync_copy(zbuf, acc.at[pl.ds(sid * RPT + j * CHUNK, CHUNK)])
        return 0
    lax.fori_loop(0, nfull, _zero, 0)
    if rem:
        pltpu.sync_copy(zbuf.at[pl.ds(0, rem)],
                        acc.at[pl.ds(sid * RPT + nfull * CHUNK, rem)])


def _load_idx(ei_hbm, which, sc0, scnt, dst):
    def _one(k, _):
        pltpu.sync_copy(ei_hbm.at[which, sc0 + k], dst.at[k])
        return 0
    lax.fori_loop(0, scnt, _one, 0)


DW = 16

@functools.partial(
    pl.kernel,
    out_type=jax.ShapeDtypeStruct((NC, N_ACC, DW), jnp.float32),
    mesh=_mesh,
    scratch_types=[
        pltpu.VMEM((MAXS, SCH, CHUNK), jnp.int32),
        pltpu.VMEM((CHUNK, DW), jnp.float32),
        pltpu.VMEM((CHUNK, DW), jnp.float32),
        pltpu.SemaphoreType.DMA,
        pltpu.VMEM_SHARED((N_ACC, DW), jnp.float32),
    ],
)
def _sc_bincount(ei_hbm, deg_out, row_idx_v, ones_v, zeros_v, asem, deg_acc):
    cid = lax.axis_index("c")
    sid = lax.axis_index("s")
    wid = sid * NC + cid
    sc0, scnt = _worker_window(wid)
    nchunks = scnt * SCH

    def _fill(i, _):
        ones_v[i] = jnp.ones((DW,), jnp.float32)
        zeros_v[i] = jnp.zeros((DW,), jnp.float32)
        return 0
    lax.fori_loop(0, CHUNK, _fill, 0)

    _load_idx(ei_hbm, 0, sc0, scnt, row_idx_v)

    _zero_acc(zeros_v, deg_acc, sid)
    plsc.subcore_barrier()

    def _accum(j, _):
        pltpu.async_copy(ones_v, deg_acc.at[row_idx_v.at[j // SCH, j % SCH]],
                         asem, add=True)
        return 0
    lax.fori_loop(0, nchunks, _accum, 0)

    def _drain(j, _):
        pltpu.make_async_copy(ones_v, deg_acc.at[row_idx_v.at[0, 0]],
                              asem).wait()
        return 0
    lax.fori_loop(0, nchunks, _drain, 0)
    plsc.subcore_barrier()

    pltpu.sync_copy(deg_acc.at[pl.ds(sid * RPT, RPT)],
                    deg_out.at[cid, pl.ds(sid * RPT, RPT)])


def _scale_mm_body(dp_ref, x_ref, w_ref, o_ref):
    deg = (dp_ref[0] + dp_ref[1])[:N]
    dis = lax.rsqrt(deg[:, :1])
    o_ref[:N] = dis * jnp.dot(x_ref[...], w_ref[...],
                              preferred_element_type=jnp.float32)


def _scale_mm(dp, x, weight):
    return pl.pallas_call(
        _scale_mm_body,
        out_shape=jax.ShapeDtypeStruct((N_ACC, D), jnp.float32),
    )(dp, x, weight)


@functools.partial(
    pl.kernel,
    out_type=jax.ShapeDtypeStruct((NC, N_ACC, D), jnp.float32),
    mesh=_mesh,
    scratch_types=[
        pltpu.VMEM((2, SCH, CHUNK), jnp.int32),
        pltpu.VMEM((2, SCH, CHUNK), jnp.int32),
        pltpu.VMEM((2, CHUNK, D), jnp.float32),
        pltpu.SemaphoreType.DMA,
        pltpu.SemaphoreType.DMA,
        pltpu.SemaphoreType.DMA,
        pltpu.VMEM_SHARED((N_ACC, D), jnp.float32),
    ],
)
def _sc_scatter(scaled_hbm, ei_hbm, out_hbm,
                row_idx_v, col_idx_v, rows_v, gsem, isem, ssem, acc):
    cid = lax.axis_index("c")
    sid = lax.axis_index("s")
    wid = sid * NC + cid
    sc0, scnt = _worker_window(wid)

    def _fill(k, _):
        for c in range(D // 16):
            rows_v[0, k, pl.ds(c * 16, 16)] = jnp.zeros((16,), jnp.float32)
        return 0
    lax.fori_loop(0, CHUNK, _fill, 0)

    nfull = RPT // CHUNK
    rem = RPT - nfull * CHUNK
    for j in range(nfull):
        pltpu.async_copy(rows_v.at[0],
                         acc.at[pl.ds(sid * RPT + j * CHUNK, CHUNK)], ssem)
    pltpu.async_copy(rows_v.at[0, pl.ds(0, rem)],
                     acc.at[pl.ds(sid * RPT + nfull * CHUNK, rem)], ssem)

    pltpu.sync_copy(ei_hbm.at[0, sc0], row_idx_v.at[0])
    pltpu.sync_copy(ei_hbm.at[1, sc0], col_idx_v.at[0])

    for j in range(nfull):
        pltpu.make_async_copy(
            rows_v.at[0], acc.at[pl.ds(sid * RPT, CHUNK)], ssem).wait()
    pltpu.make_async_copy(
        rows_v.at[0, pl.ds(0, rem)], acc.at[pl.ds(sid * RPT, rem)],
        ssem).wait()
    plsc.subcore_barrier()

    def _wait_gather(cbuf, k, rbuf):
        pltpu.make_async_copy(scaled_hbm.at[cbuf.at[k]], rbuf, gsem).wait()

    def _wait_scatter():
        pltpu.make_async_copy(rows_v.at[0], acc.at[row_idx_v.at[0, 0]],
                              ssem).wait()

    pltpu.async_copy(scaled_hbm.at[col_idx_v.at[0, 0]], rows_v.at[0], gsem)

    for s in range(MAXS):
        @pl.when(s < scnt)
        def _superchunk():
            rix, cix = row_idx_v.at[s % 2], col_idx_v.at[s % 2]

            @pl.when(s + 1 < scnt)
            def _prefetch_idx():
                pltpu.async_copy(ei_hbm.at[0, sc0 + s + 1],
                                 row_idx_v.at[(s + 1) % 2], isem)
                pltpu.async_copy(ei_hbm.at[1, sc0 + s + 1],
                                 col_idx_v.at[(s + 1) % 2], isem)

            def _chunk(k, _):
                if s == 0:
                    @pl.when(k > 0)
                    def _():
                        _wait_scatter()
                else:
                    _wait_scatter()
                pltpu.async_copy(scaled_hbm.at[cix.at[k + 1]],
                                 rows_v.at[(k + 1) % 2], gsem)
                _wait_gather(cix, k, rows_v.at[k % 2])
                pltpu.async_copy(rows_v.at[k % 2], acc.at[rix.at[k]], ssem,
                                 add=True)
                return 0
            lax.fori_loop(0, SCH - 1, _chunk, 0)

            _wait_gather(cix, SCH - 1, rows_v.at[(SCH - 1) % 2])
            pltpu.async_copy(rows_v.at[(SCH - 1) % 2],
                             acc.at[rix.at[SCH - 1]], ssem, add=True)
            _wait_scatter()

            @pl.when(s + 1 < scnt)
            def _fire_next():
                pltpu.make_async_copy(ei_hbm.at[0, sc0],
                                      row_idx_v.at[(s + 1) % 2], isem).wait()
                pltpu.make_async_copy(ei_hbm.at[1, sc0],
                                      col_idx_v.at[(s + 1) % 2], isem).wait()
                pltpu.async_copy(
                    scaled_hbm.at[col_idx_v.at[(s + 1) % 2, 0]],
                    rows_v.at[0], gsem)

    _wait_scatter()
    plsc.subcore_barrier()
    pltpu.sync_copy(acc.at[pl.ds(sid * RPT, RPT)],
                    out_hbm.at[cid, pl.ds(sid * RPT, RPT)])


def _final_body(dp_ref, ap_ref, b_ref, o_ref):
    deg = (dp_ref[0] + dp_ref[1])[:N, :1]
    dis = jnp.where(deg > 0, lax.rsqrt(deg), 0.0)
    o_ref[...] = dis * (ap_ref[0] + ap_ref[1])[:N] + b_ref[...]


def _finalize(dp, ap, bias2d):
    return pl.pallas_call(
        _final_body,
        out_shape=jax.ShapeDtypeStruct((N, D), jnp.float32),
    )(dp, ap, bias2d)


def kernel(x, edge_index, weight, bias):
    ei = edge_index.astype(jnp.int32).reshape(2, NSCH, SCH, CHUNK)

    dp = _sc_bincount(ei)
    scaled = _scale_mm(dp, x, weight)
    ap = _sc_scatter(scaled, ei)
    return _finalize(dp, ap, bias.reshape(1, D))


# speedup vs baseline: 1.1830x; 1.0125x over previous
import functools

import jax
import jax.numpy as jnp
from jax import lax
from jax.experimental import pallas as pl
from jax.experimental.pallas import tpu as pltpu
from jax.experimental.pallas import tpu_sc as plsc

N = 10000
E = 320000
D = 128

NC, NS = 2, 16
NW = NC * NS
CHUNK = 128
SCH = 20
NSCH = E // (CHUNK * SCH)
MAXS = 4
N_ACC = 10112
RPT = N_ACC // NS

_mesh = plsc.VectorSubcoreMesh(core_axis_name="c", subcore_axis_name="s")


def _worker_window(wid):
    sc0 = (wid * NSCH) // NW
    scnt = ((wid + 1) * NSCH) // NW - sc0
    return sc0, scnt


def _zero_acc(zbuf, acc, sid):
    nfull = RPT // CHUNK
    rem = RPT - nfull * CHUNK

    def _zero(j, _):
        pltpu.sync_copy(zbuf, acc.at[pl.ds(sid * RPT + j * CHUNK, CHUNK)])
        return 0
    lax.fori_loop(0, nfull, _zero, 0)
    if rem:
        pltpu.sync_copy(zbuf.at[pl.ds(0, rem)],
                        acc.at[pl.ds(sid * RPT + nfull * CHUNK, rem)])


DW = 16

@functools.partial(
    pl.kernel,
    out_type=jax.ShapeDtypeStruct((NC, N_ACC, DW), jnp.float32),
    mesh=_mesh,
    scratch_types=[
        pltpu.VMEM((MAXS, SCH, CHUNK), jnp.int32),
        pltpu.VMEM((CHUNK, DW), jnp.float32),
        pltpu.VMEM((CHUNK, DW), jnp.float32),
        pltpu.SemaphoreType.DMA,
        pltpu.SemaphoreType.DMA,
        pltpu.VMEM_SHARED((N_ACC, DW), jnp.float32),
    ],
)
def _sc_bincount(ei_hbm, deg_out, row_idx_v, ones_v, zeros_v, asem, isem,
                 deg_acc):
    cid = lax.axis_index("c")
    sid = lax.axis_index("s")
    wid = sid * NC + cid
    sc0, scnt = _worker_window(wid)
    nchunks = scnt * SCH

    def _load(k, _):
        pltpu.async_copy(ei_hbm.at[0, sc0 + k], row_idx_v.at[k], isem)
        return 0
    lax.fori_loop(0, scnt, _load, 0)

    def _fill(i, _):
        ones_v[i] = jnp.ones((DW,), jnp.float32)
        zeros_v[i] = jnp.zeros((DW,), jnp.float32)
        return 0
    lax.fori_loop(0, CHUNK, _fill, 0)

    _zero_acc(zeros_v, deg_acc, sid)

    def _ldwait(k, _):
        pltpu.make_async_copy(ei_hbm.at[0, sc0], row_idx_v.at[0], isem).wait()
        return 0
    lax.fori_loop(0, scnt, _ldwait, 0)
    plsc.subcore_barrier()

    def _accum(j, _):
        pltpu.async_copy(ones_v, deg_acc.at[row_idx_v.at[j // SCH, j % SCH]],
                         asem, add=True)
        return 0
    lax.fori_loop(0, nchunks, _accum, 0)

    def _drain(j, _):
        pltpu.make_async_copy(ones_v, deg_acc.at[row_idx_v.at[0, 0]],
                              asem).wait()
        return 0
    lax.fori_loop(0, nchunks, _drain, 0)
    plsc.subcore_barrier()

    pltpu.sync_copy(deg_acc.at[pl.ds(sid * RPT, RPT)],
                    deg_out.at[cid, pl.ds(sid * RPT, RPT)])


def _scale_mm_blk_body(dp_ref, x_ref, w_ref, o_ref):
    deg = dp_ref[0] + dp_ref[1]
    dis = lax.rsqrt(deg[:, :1])
    o_ref[...] = dis * jnp.dot(x_ref[...], w_ref[...],
                               preferred_element_type=jnp.float32)


def _scale_mm(dp, x, weight):
    blk = 1264
    return pl.pallas_call(
        _scale_mm_blk_body,
        grid=(N_ACC // blk,),
        in_specs=[
            pl.BlockSpec((NC, blk, DW), lambda i: (0, i, 0)),
            pl.BlockSpec((blk, D), lambda i: (i, 0)),
            pl.BlockSpec((D, D), lambda i: (0, 0)),
        ],
        out_specs=pl.BlockSpec((blk, D), lambda i: (i, 0)),
        out_shape=jax.ShapeDtypeStruct((N_ACC, D), jnp.float32),
    )(dp, x, weight)


@functools.partial(
    pl.kernel,
    out_type=jax.ShapeDtypeStruct((NC, N_ACC, D), jnp.float32),
    mesh=_mesh,
    scratch_types=[
        pltpu.VMEM((2, SCH, CHUNK), jnp.int32),
        pltpu.VMEM((2, SCH, CHUNK), jnp.int32),
        pltpu.VMEM((2, CHUNK, D), jnp.float32),
        pltpu.SemaphoreType.DMA,
        pltpu.SemaphoreType.DMA,
        pltpu.SemaphoreType.DMA,
        pltpu.VMEM_SHARED((N_ACC, D), jnp.float32),
    ],
)
def _sc_scatter(scaled_hbm, ei_hbm, out_hbm,
                row_idx_v, col_idx_v, rows_v, gsem, isem, ssem, acc):
    cid = lax.axis_index("c")
    sid = lax.axis_index("s")
    wid = sid * NC + cid
    sc0, scnt = _worker_window(wid)

    def _fill(k, _):
        for c in range(D // 16):
            rows_v[0, k, pl.ds(c * 16, 16)] = jnp.zeros((16,), jnp.float32)
        return 0
    lax.fori_loop(0, CHUNK, _fill, 0)

    nfull = RPT // CHUNK
    rem = RPT - nfull * CHUNK
    for j in range(nfull):
        pltpu.async_copy(rows_v.at[0],
                         acc.at[pl.ds(sid * RPT + j * CHUNK, CHUNK)], ssem)
    pltpu.async_copy(rows_v.at[0, pl.ds(0, rem)],
                     acc.at[pl.ds(sid * RPT + nfull * CHUNK, rem)], ssem)

    pltpu.sync_copy(ei_hbm.at[0, sc0], row_idx_v.at[0])
    pltpu.sync_copy(ei_hbm.at[1, sc0], col_idx_v.at[0])

    for j in range(nfull):
        pltpu.make_async_copy(
            rows_v.at[0], acc.at[pl.ds(sid * RPT, CHUNK)], ssem).wait()
    pltpu.make_async_copy(
        rows_v.at[0, pl.ds(0, rem)], acc.at[pl.ds(sid * RPT, rem)],
        ssem).wait()
    plsc.subcore_barrier()

    def _wait_gather(cbuf, k, rbuf):
        pltpu.make_async_copy(scaled_hbm.at[cbuf.at[k]], rbuf, gsem).wait()

    def _wait_scatter():
        pltpu.make_async_copy(rows_v.at[0], acc.at[row_idx_v.at[0, 0]],
                              ssem).wait()

    pltpu.async_copy(scaled_hbm.at[col_idx_v.at[0, 0]], rows_v.at[0], gsem)

    for s in range(MAXS):
        @pl.when(s < scnt)
        def _superchunk():
            rix, cix = row_idx_v.at[s % 2], col_idx_v.at[s % 2]

            @pl.when(s + 1 < scnt)
            def _prefetch_idx():
                pltpu.async_copy(ei_hbm.at[0, sc0 + s + 1],
                                 row_idx_v.at[(s + 1) % 2], isem)
                pltpu.async_copy(ei_hbm.at[1, sc0 + s + 1],
                                 col_idx_v.at[(s + 1) % 2], isem)

            def _chunk(k, _):
                if s == 0:
                    @pl.when(k > 0)
                    def _():
                        _wait_scatter()
                else:
                    _wait_scatter()
                pltpu.async_copy(scaled_hbm.at[cix.at[k + 1]],
                                 rows_v.at[(k + 1) % 2], gsem)
                _wait_gather(cix, k, rows_v.at[k % 2])
                pltpu.async_copy(rows_v.at[k % 2], acc.at[rix.at[k]], ssem,
                                 add=True)
                return 0
            lax.fori_loop(0, SCH - 1, _chunk, 0)

            _wait_gather(cix, SCH - 1, rows_v.at[(SCH - 1) % 2])
            pltpu.async_copy(rows_v.at[(SCH - 1) % 2],
                             acc.at[rix.at[SCH - 1]], ssem, add=True)
            _wait_scatter()

            @pl.when(s + 1 < scnt)
            def _fire_next():
                pltpu.make_async_copy(ei_hbm.at[0, sc0],
                                      row_idx_v.at[(s + 1) % 2], isem).wait()
                pltpu.make_async_copy(ei_hbm.at[1, sc0],
                                      col_idx_v.at[(s + 1) % 2], isem).wait()
                pltpu.async_copy(
                    scaled_hbm.at[col_idx_v.at[(s + 1) % 2, 0]],
                    rows_v.at[0], gsem)

    _wait_scatter()
    plsc.subcore_barrier()
    pltpu.sync_copy(acc.at[pl.ds(sid * RPT, RPT)],
                    out_hbm.at[cid, pl.ds(sid * RPT, RPT)])


def _final_body(dp_ref, ap_ref, b_ref, o_ref):
    deg = (dp_ref[0] + dp_ref[1])[:N, :1]
    dis = jnp.where(deg > 0, lax.rsqrt(deg), 0.0)
    o_ref[...] = dis * (ap_ref[0] + ap_ref[1])[:N] + b_ref[...]


def _finalize(dp, ap, bias2d):
    return pl.pallas_call(
        _final_body,
        out_shape=jax.ShapeDtypeStruct((N, D), jnp.float32),
    )(dp, ap, bias2d)


def kernel(x, edge_index, weight, bias):
    ei = edge_index.astype(jnp.int32).reshape(2, NSCH, SCH, CHUNK)

    dp = _sc_bincount(ei)
    scaled = _scale_mm(dp, x, weight)
    ap = _sc_scatter(scaled, ei)
    return _finalize(dp, ap, bias.reshape(1, D))


# speedup vs baseline: 1.1889x; 1.0050x over previous
import functools

import jax
import jax.numpy as jnp
from jax import lax
from jax.experimental import pallas as pl
from jax.experimental.pallas import tpu as pltpu
from jax.experimental.pallas import tpu_sc as plsc

N = 10000
E = 320000
D = 128

NC, NS = 2, 16
NW = NC * NS
CHUNK = 128
SCH = 20
NSCH = E // (CHUNK * SCH)
MAXS = 4
N_ACC = 10112
RPT = N_ACC // NS

_mesh = plsc.VectorSubcoreMesh(core_axis_name="c", subcore_axis_name="s")


def _worker_window(wid):
    sc0 = (wid * NSCH) // NW
    scnt = ((wid + 1) * NSCH) // NW - sc0
    return sc0, scnt


def _zero_acc(zbuf, acc, sid):
    nfull = RPT // CHUNK
    rem = RPT - nfull * CHUNK

    def _zero(j, _):
        pltpu.sync_copy(zbuf, acc.at[pl.ds(sid * RPT + j * CHUNK, CHUNK)])
        return 0
    lax.fori_loop(0, nfull, _zero, 0)
    if rem:
        pltpu.sync_copy(zbuf.at[pl.ds(0, rem)],
                        acc.at[pl.ds(sid * RPT + nfull * CHUNK, rem)])


DW = 16

@functools.partial(
    pl.kernel,
    out_type=jax.ShapeDtypeStruct((NC, N_ACC, DW), jnp.float32),
    mesh=_mesh,
    scratch_types=[
        pltpu.VMEM((MAXS, SCH, CHUNK), jnp.int32),
        pltpu.VMEM((CHUNK, DW), jnp.float32),
        pltpu.VMEM((CHUNK, DW), jnp.float32),
        pltpu.SemaphoreType.DMA,
        pltpu.SemaphoreType.DMA,
        pltpu.VMEM_SHARED((N_ACC, DW), jnp.float32),
    ],
)
def _sc_bincount(ei_hbm, deg_out, row_idx_v, ones_v, zeros_v, asem, isem,
                 deg_acc):
    cid = lax.axis_index("c")
    sid = lax.axis_index("s")
    wid = sid * NC + cid
    sc0, scnt = _worker_window(wid)
    nchunks = scnt * SCH

    def _load(k, _):
        pltpu.async_copy(ei_hbm.at[0, sc0 + k], row_idx_v.at[k], isem)
        return 0
    lax.fori_loop(0, scnt, _load, 0)

    def _fill(i, _):
        ones_v[i] = jnp.ones((DW,), jnp.float32)
        zeros_v[i] = jnp.zeros((DW,), jnp.float32)
        return 0
    lax.fori_loop(0, CHUNK, _fill, 0)

    _zero_acc(zeros_v, deg_acc, sid)

    def _ldwait(k, _):
        pltpu.make_async_copy(ei_hbm.at[0, sc0], row_idx_v.at[0], isem).wait()
        return 0
    lax.fori_loop(0, scnt, _ldwait, 0)
    plsc.subcore_barrier()

    def _accum(j, _):
        pltpu.async_copy(ones_v, deg_acc.at[row_idx_v.at[j // SCH, j % SCH]],
                         asem, add=True)
        return 0
    lax.fori_loop(0, nchunks, _accum, 0)

    def _drain(j, _):
        pltpu.make_async_copy(ones_v, deg_acc.at[row_idx_v.at[0, 0]],
                              asem).wait()
        return 0
    lax.fori_loop(0, nchunks, _drain, 0)
    plsc.subcore_barrier()

    pltpu.sync_copy(deg_acc.at[pl.ds(sid * RPT, RPT)],
                    deg_out.at[cid, pl.ds(sid * RPT, RPT)])


def _scale_mm_blk_body(dp_ref, x_ref, w_ref, o_ref):
    deg = dp_ref[0] + dp_ref[1]
    dis = lax.rsqrt(deg[:, :1])
    o_ref[...] = dis * jnp.dot(x_ref[...], w_ref[...],
                               preferred_element_type=jnp.float32)


def _scale_mm(dp, x, weight):
    blk = 1264
    return pl.pallas_call(
        _scale_mm_blk_body,
        grid=(N_ACC // blk,),
        in_specs=[
            pl.BlockSpec((NC, blk, DW), lambda i: (0, i, 0)),
            pl.BlockSpec((blk, D), lambda i: (i, 0)),
            pl.BlockSpec((D, D), lambda i: (0, 0)),
        ],
        out_specs=pl.BlockSpec((blk, D), lambda i: (i, 0)),
        out_shape=jax.ShapeDtypeStruct((N_ACC, D), jnp.float32),
    )(dp, x, weight)


@functools.partial(
    pl.kernel,
    out_type=jax.ShapeDtypeStruct((NC, N_ACC, D), jnp.float32),
    mesh=_mesh,
    scratch_types=[
        pltpu.VMEM((2, SCH, CHUNK), jnp.int32),
        pltpu.VMEM((2, SCH, CHUNK), jnp.int32),
        pltpu.VMEM((2, CHUNK, D), jnp.float32),
        pltpu.SemaphoreType.DMA,
        pltpu.SemaphoreType.DMA,
        pltpu.SemaphoreType.DMA,
        pltpu.VMEM_SHARED((N_ACC, D), jnp.float32),
    ],
)
def _sc_scatter(scaled_hbm, ei_hbm, out_hbm,
                row_idx_v, col_idx_v, rows_v, gsem, isem, ssem, acc):
    cid = lax.axis_index("c")
    sid = lax.axis_index("s")
    wid = sid * NC + cid
    sc0, scnt = _worker_window(wid)

    def _fill(k, _):
        for c in range(D // 16):
            rows_v[0, k, pl.ds(c * 16, 16)] = jnp.zeros((16,), jnp.float32)
        return 0
    lax.fori_loop(0, CHUNK, _fill, 0)

    nfull = RPT // CHUNK
    rem = RPT - nfull * CHUNK
    for j in range(nfull):
        pltpu.async_copy(rows_v.at[0],
                         acc.at[pl.ds(sid * RPT + j * CHUNK, CHUNK)], ssem)
    pltpu.async_copy(rows_v.at[0, pl.ds(0, rem)],
                     acc.at[pl.ds(sid * RPT + nfull * CHUNK, rem)], ssem)

    pltpu.sync_copy(ei_hbm.at[0, sc0], row_idx_v.at[0])
    pltpu.sync_copy(ei_hbm.at[1, sc0], col_idx_v.at[0])

    for j in range(nfull):
        pltpu.make_async_copy(
            rows_v.at[0], acc.at[pl.ds(sid * RPT, CHUNK)], ssem).wait()
    pltpu.make_async_copy(
        rows_v.at[0, pl.ds(0, rem)], acc.at[pl.ds(sid * RPT, rem)],
        ssem).wait()
    plsc.subcore_barrier()

    def _wait_gather(cbuf, k, rbuf):
        pltpu.make_async_copy(scaled_hbm.at[cbuf.at[k]], rbuf, gsem).wait()

    def _wait_scatter():
        pltpu.make_async_copy(rows_v.at[0], acc.at[row_idx_v.at[0, 0]],
                              ssem).wait()

    pltpu.async_copy(scaled_hbm.at[col_idx_v.at[0, 0]], rows_v.at[0], gsem)

    for s in range(MAXS):
        @pl.when(s < scnt)
        def _superchunk():
            rix, cix = row_idx_v.at[s % 2], col_idx_v.at[s % 2]

            @pl.when(s + 1 < scnt)
            def _prefetch_idx():
                pltpu.async_copy(ei_hbm.at[0, sc0 + s + 1],
                                 row_idx_v.at[(s + 1) % 2], isem)
                pltpu.async_copy(ei_hbm.at[1, sc0 + s + 1],
                                 col_idx_v.at[(s + 1) % 2], isem)

            def _chunk(k, _):
                if s == 0:
                    @pl.when(k > 0)
                    def _():
                        _wait_scatter()
                else:
                    _wait_scatter()
                pltpu.async_copy(scaled_hbm.at[cix.at[k + 1]],
                                 rows_v.at[(k + 1) % 2], gsem)
                _wait_gather(cix, k, rows_v.at[k % 2])
                pltpu.async_copy(rows_v.at[k % 2], acc.at[rix.at[k]], ssem,
                                 add=True)
                return 0
            lax.fori_loop(0, SCH - 1, _chunk, 0)

            _wait_scatter()

            @pl.when(s + 1 < scnt)
            def _fire_next():
                pltpu.make_async_copy(ei_hbm.at[0, sc0],
                                      row_idx_v.at[(s + 1) % 2], isem).wait()
                pltpu.make_async_copy(ei_hbm.at[1, sc0],
                                      col_idx_v.at[(s + 1) % 2], isem).wait()
                pltpu.async_copy(
                    scaled_hbm.at[col_idx_v.at[(s + 1) % 2, 0]],
                    rows_v.at[0], gsem)

            _wait_gather(cix, SCH - 1, rows_v.at[(SCH - 1) % 2])
            pltpu.async_copy(rows_v.at[(SCH - 1) % 2],
                             acc.at[rix.at[SCH - 1]], ssem, add=True)

    _wait_scatter()
    plsc.subcore_barrier()
    pltpu.sync_copy(acc.at[pl.ds(sid * RPT, RPT)],
                    out_hbm.at[cid, pl.ds(sid * RPT, RPT)])


def _final_body(dp_ref, ap_ref, b_ref, o_ref):
    deg = (dp_ref[0] + dp_ref[1])[:N, :1]
    dis = jnp.where(deg > 0, lax.rsqrt(deg), 0.0)
    o_ref[...] = dis * (ap_ref[0] + ap_ref[1])[:N] + b_ref[...]


def _finalize(dp, ap, bias2d):
    return pl.pallas_call(
        _final_body,
        out_shape=jax.ShapeDtypeStruct((N, D), jnp.float32),
    )(dp, ap, bias2d)


def kernel(x, edge_index, weight, bias):
    ei = edge_index.astype(jnp.int32).reshape(2, NSCH, SCH, CHUNK)

    dp = _sc_bincount(ei)
    scaled = _scale_mm(dp, x, weight)
    ap = _sc_scatter(scaled, ei)
    return _finalize(dp, ap, bias.reshape(1, D))
